# Initial kernel scaffold; baseline (speedup 1.0000x reference)
import jax
import jax.numpy as jnp
from jax.experimental import pallas as pl


def kernel(x_nchw, w1, g1, b1, w2, g2, b2):
    raise NotImplementedError("write your pallas kernel here")


# speedup vs baseline: 1.1142x; 1.1142x over previous
import functools

import jax
import jax.numpy as jnp
from jax.experimental import pallas as pl
from jax.experimental.pallas import tpu as pltpu

_EPS = 1e-5


def _shifted(a2d, s):
    c, hw = a2d.shape
    if s > 0:
        return jnp.concatenate([a2d[:, s:], jnp.zeros((c, s), a2d.dtype)], axis=1)
    if s < 0:
        return jnp.concatenate([jnp.zeros((c, -s), a2d.dtype), a2d[:, : hw + s]], axis=1)
    return a2d


def _patches(a2d, w):
    c, hw = a2d.shape
    col = jax.lax.broadcasted_iota(jnp.int32, (c, hw), 1) % w
    zero = jnp.array(0, a2d.dtype)
    blocks = []
    for dh in (-1, 0, 1):
        for dw in (-1, 0, 1):
            b = _shifted(a2d, dh * w + dw)
            if dw == -1:
                b = jnp.where(col == 0, zero, b)
            elif dw == 1:
                b = jnp.where(col == w - 1, zero, b)
            blocks.append(b)
    return jnp.concatenate(blocks, axis=0)


def _conv_stats(a_bf16, wk_ref, y_ref, st_ref, w):
    p = _patches(a_bf16, w)
    acc = jnp.dot(wk_ref[...], p, preferred_element_type=jnp.float32)
    s1 = jnp.sum(acc, axis=1, keepdims=True)
    s2 = jnp.sum(acc * acc, axis=1, keepdims=True)
    y_ref[0] = acc.astype(jnp.bfloat16)
    st_ref[0] = jnp.concatenate([s1, s2], axis=1)


def _conv1_kernel(x_ref, wk_ref, y_ref, st_ref, *, w):
    _conv_stats(x_ref[0].astype(jnp.bfloat16), wk_ref, y_ref, st_ref, w)


def _conv2_kernel(y_ref, sc_ref, sh_ref, wk_ref, o_ref, st_ref, *, w):
    a = y_ref[0].astype(jnp.float32) * sc_ref[...] + sh_ref[...]
    _conv_stats(jnp.maximum(a, 0.0).astype(jnp.bfloat16), wk_ref, o_ref, st_ref, w)


def _bnrelu_kernel(y_ref, sc_ref, sh_ref, o_ref):
    o_ref[0] = jnp.maximum(
        y_ref[0].astype(jnp.float32) * sc_ref[...] + sh_ref[...], 0.0)


def _finalize_bn(st, gamma, beta, count):
    s = jnp.sum(st, axis=0)
    mean = s[:, 0] / count
    var = s[:, 1] / count - mean * mean
    scale = gamma * jax.lax.rsqrt(var + _EPS)
    shift = beta - mean * scale
    return scale.reshape(-1, 1).astype(jnp.float32), shift.reshape(-1, 1).astype(jnp.float32)


def _parallel(n):
    return pltpu.CompilerParams(dimension_semantics=("parallel",))


def _conv1_call(x, wk, w):
    n, cin, hw = x.shape
    cout = wk.shape[0]
    return pl.pallas_call(
        functools.partial(_conv1_kernel, w=w),
        out_shape=(jax.ShapeDtypeStruct((n, cout, hw), jnp.bfloat16),
                   jax.ShapeDtypeStruct((n, cout, 2), jnp.float32)),
        grid=(n,),
        in_specs=[pl.BlockSpec((1, cin, hw), lambda i: (i, 0, 0)),
                  pl.BlockSpec((cout, 9 * cin), lambda i: (0, 0))],
        out_specs=(pl.BlockSpec((1, cout, hw), lambda i: (i, 0, 0)),
                   pl.BlockSpec((1, cout, 2), lambda i: (i, 0, 0))),
        compiler_params=_parallel(n),
    )(x, wk)


def _conv2_call(y, sc, sh, wk, w):
    n, c, hw = y.shape
    cout = wk.shape[0]
    return pl.pallas_call(
        functools.partial(_conv2_kernel, w=w),
        out_shape=(jax.ShapeDtypeStruct((n, cout, hw), jnp.bfloat16),
                   jax.ShapeDtypeStruct((n, cout, 2), jnp.float32)),
        grid=(n,),
        in_specs=[pl.BlockSpec((1, c, hw), lambda i: (i, 0, 0)),
                  pl.BlockSpec((c, 1), lambda i: (0, 0)),
                  pl.BlockSpec((c, 1), lambda i: (0, 0)),
                  pl.BlockSpec((cout, 9 * c), lambda i: (0, 0))],
        out_specs=(pl.BlockSpec((1, cout, hw), lambda i: (i, 0, 0)),
                   pl.BlockSpec((1, cout, 2), lambda i: (i, 0, 0))),
        compiler_params=_parallel(n),
    )(y, sc, sh, wk)


def _bnrelu_call(y, sc, sh):
    n, c, hw = y.shape
    return pl.pallas_call(
        _bnrelu_kernel,
        out_shape=jax.ShapeDtypeStruct((n, c, hw), jnp.float32),
        grid=(n,),
        in_specs=[pl.BlockSpec((1, c, hw), lambda i: (i, 0, 0)),
                  pl.BlockSpec((c, 1), lambda i: (0, 0)),
                  pl.BlockSpec((c, 1), lambda i: (0, 0))],
        out_specs=pl.BlockSpec((1, c, hw), lambda i: (i, 0, 0)),
        compiler_params=_parallel(n),
    )(y, sc, sh)


def _wk(w_oihw):
    cout = w_oihw.shape[0]
    return jnp.transpose(w_oihw, (0, 2, 3, 1)).reshape(cout, -1).astype(jnp.bfloat16)


@jax.jit
def _double_conv(x_nchw, w1, g1, b1, w2, g2, b2):
    n, cin, h, w = x_nchw.shape
    count = n * h * w
    x = x_nchw.reshape(n, cin, h * w)

    y1, st1 = _conv1_call(x, _wk(w1), w)
    sc1, sh1 = _finalize_bn(st1, g1, b1, count)
    y2, st2 = _conv2_call(y1, sc1, sh1, _wk(w2), w)
    sc2, sh2 = _finalize_bn(st2, g2, b2, count)
    out = _bnrelu_call(y2, sc2, sh2)
    return out.reshape(n, -1, h, w)


def kernel(x_nchw, w1, g1, b1, w2, g2, b2):
    return _double_conv(x_nchw, w1, g1, b1, w2, g2, b2)


# speedup vs baseline: 1.4494x; 1.3008x over previous
import functools

import jax
import jax.numpy as jnp
from jax.experimental import pallas as pl
from jax.experimental.pallas import tpu as pltpu

_EPS = 1e-5


def _shifted(a2d, s):
    c, hw = a2d.shape
    if s > 0:
        return jnp.concatenate([a2d[:, s:], jnp.zeros((c, s), a2d.dtype)], axis=1)
    if s < 0:
        return jnp.concatenate([jnp.zeros((c, -s), a2d.dtype), a2d[:, : hw + s]], axis=1)
    return a2d


def _patches(a2d, w):
    c, hw = a2d.shape
    col = jax.lax.broadcasted_iota(jnp.int32, (c, hw), 1) % w
    zero = jnp.array(0, a2d.dtype)
    blocks = []
    for dh in (-1, 0, 1):
        for dw in (-1, 0, 1):
            b = _shifted(a2d, dh * w + dw)
            if dw == -1:
                b = jnp.where(col == 0, zero, b)
            elif dw == 1:
                b = jnp.where(col == w - 1, zero, b)
            blocks.append(b)
    return jnp.concatenate(blocks, axis=0)


def _conv_acc(a_bf16, wk_ref, w):
    p = _patches(a_bf16, w)
    return jnp.dot(wk_ref[...], p, preferred_element_type=jnp.float32)


def _stats(acc, st_ref):
    s1 = jnp.sum(acc, axis=1, keepdims=True)
    s2 = jnp.sum(acc * acc, axis=1, keepdims=True)
    st_ref[0] = jnp.concatenate([s1, s2], axis=1)


def _conv1_kernel(x_ref, wk_ref, y_ref, st_ref, *, w):
    c = x_ref.shape[1]
    x2 = x_ref[0].astype(jnp.bfloat16).reshape(c, -1)
    acc = _conv_acc(x2, wk_ref, w)
    _stats(acc, st_ref)
    y_ref[0] = acc.astype(jnp.bfloat16)


def _bn_relu_bf16(y_ref, sc_ref, sh_ref):
    a = y_ref[0].astype(jnp.float32) * sc_ref[...] + sh_ref[...]
    return jnp.maximum(a, 0.0).astype(jnp.bfloat16)


def _conv2_stats_kernel(y_ref, sc_ref, sh_ref, wk_ref, st_ref, *, w):
    acc = _conv_acc(_bn_relu_bf16(y_ref, sc_ref, sh_ref), wk_ref, w)
    _stats(acc, st_ref)


def _conv2_out_kernel(y_ref, sc1_ref, sh1_ref, wk_ref, sc2_ref, sh2_ref, o_ref, *, w, h):
    acc = _conv_acc(_bn_relu_bf16(y_ref, sc1_ref, sh1_ref), wk_ref, w)
    out = jnp.maximum(acc * sc2_ref[...] + sh2_ref[...], 0.0)
    o_ref[0] = out.reshape(out.shape[0], h, w)


def _finalize_bn(st, gamma, beta, count):
    s = jnp.sum(st, axis=0)
    mean = s[:, 0] / count
    var = s[:, 1] / count - mean * mean
    scale = gamma * jax.lax.rsqrt(var + _EPS)
    shift = beta - mean * scale
    return scale.reshape(-1, 1).astype(jnp.float32), shift.reshape(-1, 1).astype(jnp.float32)


def _parallel():
    return pltpu.CompilerParams(dimension_semantics=("parallel",))


def _conv1_call(x, wk):
    n, cin, h, w = x.shape
    cout = wk.shape[0]
    return pl.pallas_call(
        functools.partial(_conv1_kernel, w=w),
        out_shape=(jax.ShapeDtypeStruct((n, cout, h * w), jnp.bfloat16),
                   jax.ShapeDtypeStruct((n, cout, 2), jnp.float32)),
        grid=(n,),
        in_specs=[pl.BlockSpec((1, cin, h, w), lambda i: (i, 0, 0, 0)),
                  pl.BlockSpec((cout, 9 * cin), lambda i: (0, 0))],
        out_specs=(pl.BlockSpec((1, cout, h * w), lambda i: (i, 0, 0)),
                   pl.BlockSpec((1, cout, 2), lambda i: (i, 0, 0))),
        compiler_params=_parallel(),
    )(x, wk)


def _conv2_stats_call(y, sc, sh, wk, w):
    n, c, hw = y.shape
    cout = wk.shape[0]
    return pl.pallas_call(
        functools.partial(_conv2_stats_kernel, w=w),
        out_shape=jax.ShapeDtypeStruct((n, cout, 2), jnp.float32),
        grid=(n,),
        in_specs=[pl.BlockSpec((1, c, hw), lambda i: (i, 0, 0)),
                  pl.BlockSpec((c, 1), lambda i: (0, 0)),
                  pl.BlockSpec((c, 1), lambda i: (0, 0)),
                  pl.BlockSpec((cout, 9 * c), lambda i: (0, 0))],
        out_specs=pl.BlockSpec((1, cout, 2), lambda i: (i, 0, 0)),
        compiler_params=_parallel(),
    )(y, sc, sh, wk)


def _conv2_out_call(y, sc1, sh1, wk, sc2, sh2, h, w):
    n, c, hw = y.shape
    cout = wk.shape[0]
    return pl.pallas_call(
        functools.partial(_conv2_out_kernel, w=w, h=h),
        out_shape=jax.ShapeDtypeStruct((n, cout, h, w), jnp.float32),
        grid=(n,),
        in_specs=[pl.BlockSpec((1, c, hw), lambda i: (i, 0, 0)),
                  pl.BlockSpec((c, 1), lambda i: (0, 0)),
                  pl.BlockSpec((c, 1), lambda i: (0, 0)),
                  pl.BlockSpec((cout, 9 * c), lambda i: (0, 0)),
                  pl.BlockSpec((c, 1), lambda i: (0, 0)),
                  pl.BlockSpec((c, 1), lambda i: (0, 0))],
        out_specs=pl.BlockSpec((1, cout, h, w), lambda i: (i, 0, 0, 0)),
        compiler_params=_parallel(),
    )(y, sc1, sh1, wk, sc2, sh2)


def _wk(w_oihw):
    cout = w_oihw.shape[0]
    return jnp.transpose(w_oihw, (0, 2, 3, 1)).reshape(cout, -1).astype(jnp.bfloat16)


@jax.jit
def _double_conv(x_nchw, w1, g1, b1, w2, g2, b2):
    n, cin, h, w = x_nchw.shape
    count = n * h * w

    wk1 = _wk(w1)
    wk2 = _wk(w2)
    y1, st1 = _conv1_call(x_nchw, wk1)
    sc1, sh1 = _finalize_bn(st1, g1, b1, count)
    st2 = _conv2_stats_call(y1, sc1, sh1, wk2, w)
    sc2, sh2 = _finalize_bn(st2, g2, b2, count)
    return _conv2_out_call(y1, sc1, sh1, wk2, sc2, sh2, h, w)


def kernel(x_nchw, w1, g1, b1, w2, g2, b2):
    return _double_conv(x_nchw, w1, g1, b1, w2, g2, b2)


# speedup vs baseline: 1.5496x; 1.0692x over previous
import functools

import jax
import jax.numpy as jnp
from jax.experimental import pallas as pl
from jax.experimental.pallas import tpu as pltpu

_EPS = 1e-5
_B = 4


def _shifted(a2d, s):
    c, hw = a2d.shape
    if s > 0:
        return jnp.concatenate([a2d[:, s:], jnp.zeros((c, s), a2d.dtype)], axis=1)
    if s < 0:
        return jnp.concatenate([jnp.zeros((c, -s), a2d.dtype), a2d[:, : hw + s]], axis=1)
    return a2d


def _patches(a2d, w):
    c, hw = a2d.shape
    col = jax.lax.broadcasted_iota(jnp.int32, (c, hw), 1) % w
    zero = jnp.array(0, a2d.dtype)
    blocks = []
    for dh in (-1, 0, 1):
        for dw in (-1, 0, 1):
            b = _shifted(a2d, dh * w + dw)
            if dw == -1:
                b = jnp.where(col == 0, zero, b)
            elif dw == 1:
                b = jnp.where(col == w - 1, zero, b)
            blocks.append(b)
    return jnp.concatenate(blocks, axis=0)


def _conv_acc(a_bf16, wk_ref, w):
    p = _patches(a_bf16, w)
    return jnp.dot(wk_ref[...], p, preferred_element_type=jnp.float32)


def _stats(acc):
    s1 = jnp.sum(acc, axis=1, keepdims=True)
    s2 = jnp.sum(acc * acc, axis=1, keepdims=True)
    return jnp.concatenate([s1, s2], axis=1)


def _conv1_kernel(x_ref, wk_ref, y_ref, st_ref, *, w):
    c = x_ref.shape[1]
    st = None
    for b in range(_B):
        x2 = x_ref[b].astype(jnp.bfloat16).reshape(c, -1)
        acc = _conv_acc(x2, wk_ref, w)
        st = _stats(acc) if st is None else st + _stats(acc)
        y_ref[b] = acc.astype(jnp.bfloat16)
    st_ref[0] = st


def _bn_relu_bf16(y, sc_ref, sh_ref):
    return jnp.maximum(y * sc_ref[...] + sh_ref[...], jnp.bfloat16(0))


def _conv2_stats_kernel(y_ref, sc_ref, sh_ref, wk_ref, st_ref, *, w):
    st = None
    for b in range(_B):
        acc = _conv_acc(_bn_relu_bf16(y_ref[b], sc_ref, sh_ref), wk_ref, w)
        st = _stats(acc) if st is None else st + _stats(acc)
    st_ref[0] = st


def _conv2_out_kernel(y_ref, sc1_ref, sh1_ref, wk_ref, sc2_ref, sh2_ref, o_ref, *, w, h):
    for b in range(_B):
        acc = _conv_acc(_bn_relu_bf16(y_ref[b], sc1_ref, sh1_ref), wk_ref, w)
        out = jnp.maximum(acc * sc2_ref[...] + sh2_ref[...], 0.0)
        o_ref[b] = out.reshape(out.shape[0], h, w)


def _finalize_bn(st, gamma, beta, count, out_dtype):
    s = jnp.sum(st, axis=0)
    mean = s[:, 0] / count
    var = s[:, 1] / count - mean * mean
    scale = gamma * jax.lax.rsqrt(var + _EPS)
    shift = beta - mean * scale
    return scale.reshape(-1, 1).astype(out_dtype), shift.reshape(-1, 1).astype(out_dtype)


def _parallel():
    return pltpu.CompilerParams(dimension_semantics=("parallel",))


def _conv1_call(x, wk):
    n, cin, h, w = x.shape
    cout = wk.shape[0]
    return pl.pallas_call(
        functools.partial(_conv1_kernel, w=w),
        out_shape=(jax.ShapeDtypeStruct((n, cout, h * w), jnp.bfloat16),
                   jax.ShapeDtypeStruct((n // _B, cout, 2), jnp.float32)),
        grid=(n // _B,),
        in_specs=[pl.BlockSpec((_B, cin, h, w), lambda i: (i, 0, 0, 0)),
                  pl.BlockSpec((cout, 9 * cin), lambda i: (0, 0))],
        out_specs=(pl.BlockSpec((_B, cout, h * w), lambda i: (i, 0, 0)),
                   pl.BlockSpec((1, cout, 2), lambda i: (i, 0, 0))),
        compiler_params=_parallel(),
    )(x, wk)


def _conv2_stats_call(y, sc, sh, wk, w):
    n, c, hw = y.shape
    cout = wk.shape[0]
    return pl.pallas_call(
        functools.partial(_conv2_stats_kernel, w=w),
        out_shape=jax.ShapeDtypeStruct((n // _B, cout, 2), jnp.float32),
        grid=(n // _B,),
        in_specs=[pl.BlockSpec((_B, c, hw), lambda i: (i, 0, 0)),
                  pl.BlockSpec((c, 1), lambda i: (0, 0)),
                  pl.BlockSpec((c, 1), lambda i: (0, 0)),
                  pl.BlockSpec((cout, 9 * c), lambda i: (0, 0))],
        out_specs=pl.BlockSpec((1, cout, 2), lambda i: (i, 0, 0)),
        compiler_params=_parallel(),
    )(y, sc, sh, wk)


def _conv2_out_call(y, sc1, sh1, wk, sc2, sh2, h, w):
    n, c, hw = y.shape
    cout = wk.shape[0]
    return pl.pallas_call(
        functools.partial(_conv2_out_kernel, w=w, h=h),
        out_shape=jax.ShapeDtypeStruct((n, cout, h, w), jnp.float32),
        grid=(n // _B,),
        in_specs=[pl.BlockSpec((_B, c, hw), lambda i: (i, 0, 0)),
                  pl.BlockSpec((c, 1), lambda i: (0, 0)),
                  pl.BlockSpec((c, 1), lambda i: (0, 0)),
                  pl.BlockSpec((cout, 9 * c), lambda i: (0, 0)),
                  pl.BlockSpec((c, 1), lambda i: (0, 0)),
                  pl.BlockSpec((c, 1), lambda i: (0, 0))],
        out_specs=pl.BlockSpec((_B, cout, h, w), lambda i: (i, 0, 0, 0)),
        compiler_params=_parallel(),
    )(y, sc1, sh1, wk, sc2, sh2)


def _wk(w_oihw):
    cout = w_oihw.shape[0]
    return jnp.transpose(w_oihw, (0, 2, 3, 1)).reshape(cout, -1).astype(jnp.bfloat16)


@jax.jit
def _double_conv(x_nchw, w1, g1, b1, w2, g2, b2):
    n, cin, h, w = x_nchw.shape
    count = n * h * w

    wk1 = _wk(w1)
    wk2 = _wk(w2)
    y1, st1 = _conv1_call(x_nchw, wk1)
    sc1, sh1 = _finalize_bn(st1, g1, b1, count, jnp.bfloat16)
    st2 = _conv2_stats_call(y1, sc1, sh1, wk2, w)
    sc2, sh2 = _finalize_bn(st2, g2, b2, count, jnp.float32)
    return _conv2_out_call(y1, sc1, sh1, wk2, sc2, sh2, h, w)


def kernel(x_nchw, w1, g1, b1, w2, g2, b2):
    return _double_conv(x_nchw, w1, g1, b1, w2, g2, b2)


# speedup vs baseline: 1.6056x; 1.0361x over previous
import functools

import jax
import jax.numpy as jnp
from jax.experimental import pallas as pl
from jax.experimental.pallas import tpu as pltpu

_EPS = 1e-5
_B = 4


def _shifted(a2d, s):
    c, hw = a2d.shape
    if s > 0:
        return jnp.concatenate([a2d[:, s:], jnp.zeros((c, s), a2d.dtype)], axis=1)
    if s < 0:
        return jnp.concatenate([jnp.zeros((c, -s), a2d.dtype), a2d[:, : hw + s]], axis=1)
    return a2d


def _patches(a2d, w):
    c, hw = a2d.shape
    col = jax.lax.broadcasted_iota(jnp.int32, (c, hw), 1) % w
    zero = jnp.array(0, a2d.dtype)
    blocks = []
    for dh in (-1, 0, 1):
        for dw in (-1, 0, 1):
            b = _shifted(a2d, dh * w + dw)
            if dw == -1:
                b = jnp.where(col == 0, zero, b)
            elif dw == 1:
                b = jnp.where(col == w - 1, zero, b)
            blocks.append(b)
    return jnp.concatenate(blocks, axis=0)


def _conv_acc(a_bf16, wk_ref, w):
    p = _patches(a_bf16, w)
    return jnp.dot(wk_ref[...], p, preferred_element_type=jnp.float32)


def _stats(acc):
    s1 = jnp.sum(acc, axis=1, keepdims=True)
    s2 = jnp.sum(acc * acc, axis=1, keepdims=True)
    return jnp.concatenate([s1, s2], axis=1)


def _affine_from_stats(st, g_ref, b_ref, count):
    mean = st[:, 0:1] / count
    var = st[:, 1:2] / count - mean * mean
    scale = g_ref[...] * jax.lax.rsqrt(var + _EPS)
    shift = b_ref[...] - mean * scale
    return scale, shift


def _fused_kernel(x_ref, wk1_ref, g1_ref, b1_ref, wk2_ref, g2_ref, b2_ref,
                  o_ref, ybuf, st, aff1, aff2, *, nb, w, h, count):
    i = pl.program_id(0)
    c = x_ref.shape[1]

    @pl.when(i == 0)
    def _init():
        st[...] = jnp.zeros_like(st)

    @pl.when(i < nb)
    def _phase_a():
        acc_st = st[...]
        for b in range(_B):
            x2 = x_ref[b].astype(jnp.bfloat16).reshape(c, -1)
            acc = _conv_acc(x2, wk1_ref, w)
            acc_st = acc_st + _stats(acc)
            ybuf[i * _B + b] = acc.astype(jnp.bfloat16)
        st[...] = acc_st

    @pl.when(i == nb - 1)
    def _finalize_bn1():
        scale, shift = _affine_from_stats(st[...], g1_ref, b1_ref, count)
        aff1[0] = scale.astype(jnp.bfloat16)
        aff1[1] = shift.astype(jnp.bfloat16)
        st[...] = jnp.zeros_like(st)

    @pl.when(jnp.logical_and(i >= nb, i < 2 * nb))
    def _phase_b():
        j = i - nb
        sc1 = aff1[0]
        sh1 = aff1[1]
        acc_st = st[...]
        for b in range(_B):
            y = ybuf[j * _B + b]
            a = jnp.maximum(y * sc1 + sh1, jnp.bfloat16(0))
            ybuf[j * _B + b] = a
            acc = _conv_acc(a, wk2_ref, w)
            acc_st = acc_st + _stats(acc)
        st[...] = acc_st

    @pl.when(i == 2 * nb - 1)
    def _finalize_bn2():
        scale, shift = _affine_from_stats(st[...], g2_ref, b2_ref, count)
        aff2[0] = scale
        aff2[1] = shift

    @pl.when(i >= 2 * nb)
    def _phase_c():
        j = i - 2 * nb
        sc2 = aff2[0]
        sh2 = aff2[1]
        for b in range(_B):
            acc = _conv_acc(ybuf[j * _B + b], wk2_ref, w)
            out = jnp.maximum(acc * sc2 + sh2, 0.0)
            o_ref[b] = out.reshape(out.shape[0], h, w)


def _wk(w_oihw):
    cout = w_oihw.shape[0]
    return jnp.transpose(w_oihw, (0, 2, 3, 1)).reshape(cout, -1).astype(jnp.bfloat16)


@jax.jit
def _double_conv(x_nchw, w1, g1, b1, w2, g2, b2):
    n, cin, h, w = x_nchw.shape
    cout = w1.shape[0]
    hw = h * w
    nb = n // _B
    count = float(n * hw)

    col2 = lambda v: v.reshape(-1, 1).astype(jnp.float32)
    const2 = lambda i: (0, 0)

    out = pl.pallas_call(
        functools.partial(_fused_kernel, nb=nb, w=w, h=h, count=count),
        out_shape=jax.ShapeDtypeStruct((n, cout, h, w), jnp.float32),
        grid=(3 * nb,),
        in_specs=[
            pl.BlockSpec((_B, cin, h, w),
                         lambda i: (jnp.minimum(i, nb - 1), 0, 0, 0)),
            pl.BlockSpec((cout, 9 * cin), const2),
            pl.BlockSpec((cout, 1), const2),
            pl.BlockSpec((cout, 1), const2),
            pl.BlockSpec((cout, 9 * cout), const2),
            pl.BlockSpec((cout, 1), const2),
            pl.BlockSpec((cout, 1), const2),
        ],
        out_specs=pl.BlockSpec((_B, cout, h, w),
                               lambda i: (jnp.maximum(i - 2 * nb, 0), 0, 0, 0)),
        scratch_shapes=[
            pltpu.VMEM((n, cout, hw), jnp.bfloat16),
            pltpu.VMEM((cout, 2), jnp.float32),
            pltpu.VMEM((2, cout, 1), jnp.bfloat16),
            pltpu.VMEM((2, cout, 1), jnp.float32),
        ],
        compiler_params=pltpu.CompilerParams(
            dimension_semantics=("arbitrary",)),
    )(x_nchw, _wk(w1), col2(g1), col2(b1), _wk(w2), col2(g2), col2(b2))
    return out


def kernel(x_nchw, w1, g1, b1, w2, g2, b2):
    return _double_conv(x_nchw, w1, g1, b1, w2, g2, b2)


# speedup vs baseline: 2.1920x; 1.3652x over previous
import functools

import jax
import jax.numpy as jnp
from jax.experimental import pallas as pl
from jax.experimental.pallas import tpu as pltpu

_EPS = 1e-5
_B = 4


def _shifted(a2d, s):
    c, hw = a2d.shape
    if s > 0:
        return jnp.concatenate([a2d[:, s:], jnp.zeros((c, s), a2d.dtype)], axis=1)
    if s < 0:
        return jnp.concatenate([jnp.zeros((c, -s), a2d.dtype), a2d[:, : hw + s]], axis=1)
    return a2d


def _patches(a2d, w):
    c, hw = a2d.shape
    col = jax.lax.broadcasted_iota(jnp.int32, (c, hw), 1) % w
    zero = jnp.array(0, a2d.dtype)
    blocks = []
    for dh in (-1, 0, 1):
        for dw in (-1, 0, 1):
            b = _shifted(a2d, dh * w + dw)
            if dw == -1:
                b = jnp.where(col == 0, zero, b)
            elif dw == 1:
                b = jnp.where(col == w - 1, zero, b)
            blocks.append(b)
    return jnp.concatenate(blocks, axis=0)


def _conv_acc(a_bf16, wk_ref, w):
    p = _patches(a_bf16, w)
    return jnp.dot(wk_ref[...], p, preferred_element_type=jnp.float32)


def _stats(acc):
    s1 = jnp.sum(acc, axis=1, keepdims=True)
    s2 = jnp.sum(acc * acc, axis=1, keepdims=True)
    return jnp.concatenate([s1, s2], axis=1)


def _affine_from_stats(st, g_ref, b_ref, count):
    mean = st[:, 0:1] / count
    var = st[:, 1:2] / count - mean * mean
    scale = g_ref[...] * jax.lax.rsqrt(var + _EPS)
    shift = b_ref[...] - mean * scale
    return scale, shift


def _fused_kernel(x_ref, wk1_ref, g1_ref, b1_ref, wk2_ref, g2_ref, b2_ref,
                  o_ref, ybuf, st, aff1, aff2, *, nb, w, h, count):
    i = pl.program_id(0)
    c = x_ref.shape[1]

    @pl.when(i == 0)
    def _init():
        st[...] = jnp.zeros_like(st)

    @pl.when(i < nb)
    def _phase_a():
        acc_st = st[...]
        xf = x_ref[...].astype(jnp.bfloat16).reshape(_B * c, h * w)
        for b in range(_B):
            x2 = xf[b * c:(b + 1) * c]
            acc = _conv_acc(x2, wk1_ref, w)
            acc_st = acc_st + _stats(acc)
            ybuf[i * _B + b] = acc.astype(jnp.bfloat16)
        st[...] = acc_st

    @pl.when(i == nb - 1)
    def _finalize_bn1():
        scale, shift = _affine_from_stats(st[...], g1_ref, b1_ref, count)
        aff1[0] = scale.astype(jnp.bfloat16)
        aff1[1] = shift.astype(jnp.bfloat16)
        st[...] = jnp.zeros_like(st)

    @pl.when(jnp.logical_and(i >= nb, i < 2 * nb))
    def _phase_b():
        j = i - nb
        sc1 = aff1[0]
        sh1 = aff1[1]
        acc_st = st[...]
        for b in range(_B):
            y = ybuf[j * _B + b]
            a = jnp.maximum(y * sc1 + sh1, jnp.bfloat16(0))
            acc = _conv_acc(a, wk2_ref, w)
            acc_st = acc_st + _stats(acc)
            ybuf[j * _B + b] = acc.astype(jnp.bfloat16)
        st[...] = acc_st

    @pl.when(i == 2 * nb - 1)
    def _finalize_bn2():
        scale, shift = _affine_from_stats(st[...], g2_ref, b2_ref, count)
        aff2[0] = scale
        aff2[1] = shift

    @pl.when(i >= 2 * nb)
    def _phase_c():
        j = i - 2 * nb
        sc2 = aff2[0]
        sh2 = aff2[1]
        for b in range(_B):
            acc = ybuf[j * _B + b]
            out = jnp.maximum(acc * sc2 + sh2, 0.0)
            o_ref[b] = out.reshape(out.shape[0], h, w)


def _wk(w_oihw):
    cout = w_oihw.shape[0]
    return jnp.transpose(w_oihw, (0, 2, 3, 1)).reshape(cout, -1).astype(jnp.bfloat16)


@jax.jit
def _double_conv(x_nchw, w1, g1, b1, w2, g2, b2):
    n, cin, h, w = x_nchw.shape
    cout = w1.shape[0]
    hw = h * w
    nb = n // _B
    count = float(n * hw)

    col2 = lambda v: v.reshape(-1, 1).astype(jnp.float32)
    const2 = lambda i: (0, 0)

    out = pl.pallas_call(
        functools.partial(_fused_kernel, nb=nb, w=w, h=h, count=count),
        out_shape=jax.ShapeDtypeStruct((n, cout, h, w), jnp.float32),
        grid=(3 * nb,),
        in_specs=[
            pl.BlockSpec((_B, cin, h, w),
                         lambda i: (jnp.minimum(i, nb - 1), 0, 0, 0)),
            pl.BlockSpec((cout, 9 * cin), const2),
            pl.BlockSpec((cout, 1), const2),
            pl.BlockSpec((cout, 1), const2),
            pl.BlockSpec((cout, 9 * cout), const2),
            pl.BlockSpec((cout, 1), const2),
            pl.BlockSpec((cout, 1), const2),
        ],
        out_specs=pl.BlockSpec((_B, cout, h, w),
                               lambda i: (jnp.maximum(i - 2 * nb, 0), 0, 0, 0)),
        scratch_shapes=[
            pltpu.VMEM((n, cout, hw), jnp.bfloat16),
            pltpu.VMEM((cout, 2), jnp.float32),
            pltpu.VMEM((2, cout, 1), jnp.bfloat16),
            pltpu.VMEM((2, cout, 1), jnp.float32),
        ],
        compiler_params=pltpu.CompilerParams(
            dimension_semantics=("arbitrary",)),
    )(x_nchw, _wk(w1), col2(g1), col2(b1), _wk(w2), col2(g2), col2(b2))
    return out


def kernel(x_nchw, w1, g1, b1, w2, g2, b2):
    return _double_conv(x_nchw, w1, g1, b1, w2, g2, b2)


# speedup vs baseline: 2.2868x; 1.0433x over previous
import functools

import jax
import jax.numpy as jnp
from jax.experimental import pallas as pl
from jax.experimental.pallas import tpu as pltpu

_EPS = 1e-5
_B = 4


def _shifted(a2d, s):
    c, hw = a2d.shape
    if s > 0:
        return jnp.concatenate([a2d[:, s:], jnp.zeros((c, s), a2d.dtype)], axis=1)
    if s < 0:
        return jnp.concatenate([jnp.zeros((c, -s), a2d.dtype), a2d[:, : hw + s]], axis=1)
    return a2d


def _patches(a2d, w):
    c, hw = a2d.shape
    col = jax.lax.broadcasted_iota(jnp.int32, (c, hw), 1) % w
    zero = jnp.array(0, a2d.dtype)
    blocks = []
    for dh in (-1, 0, 1):
        for dw in (-1, 0, 1):
            b = _shifted(a2d, dh * w + dw)
            if dw == -1:
                b = jnp.where(col == 0, zero, b)
            elif dw == 1:
                b = jnp.where(col == w - 1, zero, b)
            blocks.append(b)
    return jnp.concatenate(blocks, axis=0)


def _conv_acc(a_bf16, wk_ref, w):
    p = _patches(a_bf16, w)
    return jnp.dot(wk_ref[...], p, preferred_element_type=jnp.float32)


def _stats(acc):
    s1 = jnp.sum(acc, axis=1, keepdims=True)
    s2 = jnp.sum(acc * acc, axis=1, keepdims=True)
    return jnp.concatenate([s1, s2], axis=1)


def _affine_from_stats(st, g_ref, b_ref, count):
    mean = st[:, 0:1] / count
    var = st[:, 1:2] / count - mean * mean
    scale = g_ref[...] * jax.lax.rsqrt(var + _EPS)
    shift = b_ref[...] - mean * scale
    return scale, shift


def _fused_kernel(x_ref, wks_ref, gb_ref, o_ref, ybuf, st, aff1, aff2,
                  *, nb, w, h, count):
    i = pl.program_id(0)
    c = x_ref.shape[1]
    cout = wks_ref.shape[0] // 2
    wk1_ref = wks_ref.at[:cout]
    wk2_ref = wks_ref.at[cout:]
    g1_ref = gb_ref.at[:, 0:1]
    b1_ref = gb_ref.at[:, 1:2]
    g2_ref = gb_ref.at[:, 2:3]
    b2_ref = gb_ref.at[:, 3:4]

    @pl.when(i == 0)
    def _init():
        st[...] = jnp.zeros_like(st)

    @pl.when(i < nb)
    def _phase_a():
        acc_st = st[...]
        xf = x_ref[...].astype(jnp.bfloat16).reshape(_B * c, h * w)
        for b in range(_B):
            x2 = xf[b * c:(b + 1) * c]
            acc = _conv_acc(x2, wk1_ref, w)
            acc_st = acc_st + _stats(acc)
            ybuf[i * _B + b] = acc.astype(jnp.bfloat16)
        st[...] = acc_st

    @pl.when(i == nb - 1)
    def _finalize_bn1():
        scale, shift = _affine_from_stats(st[...], g1_ref, b1_ref, count)
        aff1[0] = scale.astype(jnp.bfloat16)
        aff1[1] = shift.astype(jnp.bfloat16)
        st[...] = jnp.zeros_like(st)

    @pl.when(jnp.logical_and(i >= nb, i < 2 * nb))
    def _phase_b():
        j = i - nb
        sc1 = aff1[0]
        sh1 = aff1[1]
        acc_st = st[...]
        for b in range(_B):
            y = ybuf[j * _B + b]
            a = jnp.maximum(y * sc1 + sh1, jnp.bfloat16(0))
            acc = _conv_acc(a, wk2_ref, w)
            acc_st = acc_st + _stats(acc)
            ybuf[j * _B + b] = acc.astype(jnp.bfloat16)
        st[...] = acc_st

    @pl.when(i == 2 * nb - 1)
    def _finalize_bn2():
        scale, shift = _affine_from_stats(st[...], g2_ref, b2_ref, count)
        aff2[0] = scale
        aff2[1] = shift

    @pl.when(i >= 2 * nb)
    def _phase_c():
        j = i - 2 * nb
        sc2 = aff2[0]
        sh2 = aff2[1]
        for b in range(_B):
            acc = ybuf[j * _B + b]
            out = jnp.maximum(acc * sc2 + sh2, 0.0)
            o_ref[b] = out.reshape(out.shape[0], h, w)


def _wk(w_oihw):
    cout = w_oihw.shape[0]
    return jnp.transpose(w_oihw, (0, 2, 3, 1)).reshape(cout, -1).astype(jnp.bfloat16)


@jax.jit
def _double_conv(x_nchw, w1, g1, b1, w2, g2, b2):
    n, cin, h, w = x_nchw.shape
    cout = w1.shape[0]
    hw = h * w
    nb = n // _B
    count = float(n * hw)

    col = lambda v: v.reshape(-1, 1).astype(jnp.float32)
    const2 = lambda i: (0, 0)

    wks = jnp.concatenate([_wk(w1), _wk(w2)], axis=0)
    gb = jnp.concatenate([col(g1), col(b1), col(g2), col(b2)], axis=1)

    out = pl.pallas_call(
        functools.partial(_fused_kernel, nb=nb, w=w, h=h, count=count),
        out_shape=jax.ShapeDtypeStruct((n, cout, h, w), jnp.float32),
        grid=(3 * nb,),
        in_specs=[
            pl.BlockSpec((_B, cin, h, w),
                         lambda i: (jnp.minimum(i, nb - 1), 0, 0, 0)),
            pl.BlockSpec((2 * cout, 9 * cin), const2),
            pl.BlockSpec((cout, 4), const2),
        ],
        out_specs=pl.BlockSpec((_B, cout, h, w),
                               lambda i: (jnp.maximum(i - 2 * nb, 0), 0, 0, 0)),
        scratch_shapes=[
            pltpu.VMEM((n, cout, hw), jnp.bfloat16),
            pltpu.VMEM((cout, 2), jnp.float32),
            pltpu.VMEM((2, cout, 1), jnp.bfloat16),
            pltpu.VMEM((2, cout, 1), jnp.float32),
        ],
        compiler_params=pltpu.CompilerParams(
            dimension_semantics=("arbitrary",)),
    )(x_nchw, wks, gb)
    return out


def kernel(x_nchw, w1, g1, b1, w2, g2, b2):
    return _double_conv(x_nchw, w1, g1, b1, w2, g2, b2)


# speedup vs baseline: 2.2888x; 1.0009x over previous
import functools

import jax
import jax.numpy as jnp
from jax.experimental import pallas as pl
from jax.experimental.pallas import tpu as pltpu

_EPS = 1e-5
_B = 4


def _shifted(a2d, s):
    c, hw = a2d.shape
    if s > 0:
        return jnp.concatenate([a2d[:, s:], jnp.zeros((c, s), a2d.dtype)], axis=1)
    if s < 0:
        return jnp.concatenate([jnp.zeros((c, -s), a2d.dtype), a2d[:, : hw + s]], axis=1)
    return a2d


def _patches(a2d, w):
    c, hw = a2d.shape
    col = jax.lax.broadcasted_iota(jnp.int32, (c, hw), 1) % w
    zero = jnp.array(0, a2d.dtype)
    blocks = []
    for dh in (-1, 0, 1):
        for dw in (-1, 0, 1):
            b = _shifted(a2d, dh * w + dw)
            if dw == -1:
                b = jnp.where(col == 0, zero, b)
            elif dw == 1:
                b = jnp.where(col == w - 1, zero, b)
            blocks.append(b)
    return jnp.concatenate(blocks, axis=0)


def _conv_acc(a_bf16, wk_ref, w):
    p = _patches(a_bf16, w)
    return jnp.dot(wk_ref[...], p, preferred_element_type=jnp.float32)


def _stats(acc):
    s1 = jnp.sum(acc, axis=1, keepdims=True)
    s2 = jnp.sum(acc * acc, axis=1, keepdims=True)
    return jnp.concatenate([s1, s2], axis=1)


def _affine_from_stats(st, g_ref, b_ref, count):
    mean = st[:, 0:1] / count
    var = st[:, 1:2] / count - mean * mean
    scale = g_ref[...] * jax.lax.rsqrt(var + _EPS)
    shift = b_ref[...] - mean * scale
    return scale, shift


def _fused_kernel(x_ref, wks_ref, gb_ref, o_ref, ybuf, st, aff1, aff2,
                  *, nb, w, h, count):
    i = pl.program_id(0)
    c = x_ref.shape[1]
    cout = wks_ref.shape[0] // 2
    wk1_ref = wks_ref.at[:cout]
    wk2_ref = wks_ref.at[cout:]
    g1_ref = gb_ref.at[:, 0:1]
    b1_ref = gb_ref.at[:, 1:2]
    g2_ref = gb_ref.at[:, 2:3]
    b2_ref = gb_ref.at[:, 3:4]

    @pl.when(i == 0)
    def _init():
        st[...] = jnp.zeros_like(st)

    @pl.when(i < nb)
    def _phase_a():
        acc_st = st[...]
        xf = x_ref[...].astype(jnp.bfloat16).reshape(_B * c, h * w)
        for b in range(_B):
            x2 = xf[b * c:(b + 1) * c]
            acc = _conv_acc(x2, wk1_ref, w)
            acc_st = acc_st + _stats(acc)
            ybuf[i * _B + b] = acc.astype(jnp.bfloat16)
        st[...] = acc_st

    @pl.when(i == nb - 1)
    def _finalize_bn1():
        scale, shift = _affine_from_stats(st[...], g1_ref, b1_ref, count)
        aff1[0] = scale
        aff1[1] = shift
        st[...] = jnp.zeros_like(st)

    @pl.when(jnp.logical_and(i >= nb, i < 2 * nb))
    def _phase_b():
        j = i - nb
        sc1 = aff1[0]
        sh1 = aff1[1]
        acc_st = st[...]
        for b in range(_B):
            y = ybuf[j * _B + b]
            a = jnp.maximum(y.astype(jnp.float32) * sc1 + sh1, 0.0).astype(jnp.bfloat16)
            acc = _conv_acc(a, wk2_ref, w)
            acc_st = acc_st + _stats(acc)
            ybuf[j * _B + b] = acc.astype(jnp.bfloat16)
        st[...] = acc_st

    @pl.when(i == 2 * nb - 1)
    def _finalize_bn2():
        scale, shift = _affine_from_stats(st[...], g2_ref, b2_ref, count)
        aff2[0] = scale
        aff2[1] = shift

    @pl.when(i >= 2 * nb)
    def _phase_c():
        j = i - 2 * nb
        sc2 = aff2[0]
        sh2 = aff2[1]
        for b in range(_B):
            acc = ybuf[j * _B + b]
            out = jnp.maximum(acc * sc2 + sh2, 0.0)
            o_ref[b] = out.reshape(out.shape[0], h, w)


def _wk(w_oihw):
    cout = w_oihw.shape[0]
    return jnp.transpose(w_oihw, (0, 2, 3, 1)).reshape(cout, -1).astype(jnp.bfloat16)


@jax.jit
def _double_conv(x_nchw, w1, g1, b1, w2, g2, b2):
    n, cin, h, w = x_nchw.shape
    cout = w1.shape[0]
    hw = h * w
    nb = n // _B
    count = float(n * hw)

    col = lambda v: v.reshape(-1, 1).astype(jnp.float32)
    const2 = lambda i: (0, 0)

    wks = jnp.concatenate([_wk(w1), _wk(w2)], axis=0)
    gb = jnp.concatenate([col(g1), col(b1), col(g2), col(b2)], axis=1)

    out = pl.pallas_call(
        functools.partial(_fused_kernel, nb=nb, w=w, h=h, count=count),
        out_shape=jax.ShapeDtypeStruct((n, cout, h, w), jnp.float32),
        grid=(3 * nb,),
        in_specs=[
            pl.BlockSpec((_B, cin, h, w),
                         lambda i: (jnp.minimum(i, nb - 1), 0, 0, 0)),
            pl.BlockSpec((2 * cout, 9 * cin), const2),
            pl.BlockSpec((cout, 4), const2),
        ],
        out_specs=pl.BlockSpec((_B, cout, h, w),
                               lambda i: (jnp.maximum(i - 2 * nb, 0), 0, 0, 0)),
        scratch_shapes=[
            pltpu.VMEM((n, cout, hw), jnp.bfloat16),
            pltpu.VMEM((cout, 2), jnp.float32),
            pltpu.VMEM((2, cout, 1), jnp.float32),
            pltpu.VMEM((2, cout, 1), jnp.float32),
        ],
        compiler_params=pltpu.CompilerParams(
            dimension_semantics=("arbitrary",)),
    )(x_nchw, wks, gb)
    return out


def kernel(x_nchw, w1, g1, b1, w2, g2, b2):
    return _double_conv(x_nchw, w1, g1, b1, w2, g2, b2)
